# baseline (speedup 1.0000x reference)
import functools

import jax
import jax.numpy as jnp
from jax import lax
from jax.experimental import pallas as pl
from jax.experimental.pallas import tpu as pltpu
from jax.experimental.pallas import tpu_sc as plsc


def _tc_stage(x_r, e, total_count):
    B, C, HW = x_r.shape
    K = e.shape[1]
    inv_count = 1.0 / total_count

    def body(x_ref, e_ref, idx_ref, loss_ref, acc_ref):
        i = pl.program_id(0)
        xb = x_ref[0]
        et = e_ref[...]
        x2 = jnp.sum(xb * xb, axis=0)
        e2 = jnp.sum(et * et, axis=0)
        xe = lax.dot_general(
            xb, et, (((0,), (0,)), ((), ())),
            preferred_element_type=jnp.float32)
        scores = (x2[:, None] - 2.0 * xe) + e2[None, :]
        mins = jnp.min(scores, axis=1)
        idx = jnp.argmin(scores, axis=1).astype(jnp.int32)
        idx_ref[0, 0, :] = idx

        @pl.when(i == 0)
        def _():
            acc_ref[...] = jnp.zeros_like(acc_ref)

        acc_ref[...] += mins.reshape(acc_ref.shape)

        @pl.when(i == pl.num_programs(0) - 1)
        def _():
            loss_ref[0, 0] = jnp.sum(acc_ref[...]) * inv_count

    return pl.pallas_call(
        body,
        grid=(B,),
        in_specs=[
            pl.BlockSpec((1, C, HW), lambda i: (i, 0, 0)),
            pl.BlockSpec((C, K), lambda i: (0, 0)),
        ],
        out_specs=[
            pl.BlockSpec((1, 1, HW), lambda i: (i, 0, 0)),
            pl.BlockSpec(block_shape=(1, 1), index_map=lambda i: (0, 0),
                         memory_space=pltpu.SMEM),
        ],
        out_shape=[
            jax.ShapeDtypeStruct((B, 1, HW), jnp.int32),
            jax.ShapeDtypeStruct((1, 1), jnp.float32),
        ],
        scratch_shapes=[pltpu.VMEM((8, HW // 8), jnp.float32)],
        compiler_params=pltpu.CompilerParams(
            dimension_semantics=("arbitrary",)),
    )(x_r, e)


def _sc_gather(table, idx2d):
    K, C = table.shape
    R, CH = idx2d.shape
    info = plsc.get_sparse_core_info()
    NW = info.num_cores * info.num_subcores
    nch = R // NW
    NB = min(3, nch)

    mesh = plsc.VectorSubcoreMesh(core_axis_name="c", subcore_axis_name="s")

    @functools.partial(
        pl.kernel,
        mesh=mesh,
        out_type=jax.ShapeDtypeStruct((R * CH, C), jnp.float32),
        scratch_types=[
            pltpu.VMEM((nch, CH), jnp.int32),
        ]
        + [pltpu.VMEM((CH, C), jnp.float32) for _ in range(NB)]
        + [pltpu.SemaphoreType.DMA for _ in range(2 * NB)],
    )
    def k(table_hbm, idx_hbm, out_hbm, idx_v, *rest):
        bufs = rest[:NB]
        gsems = rest[NB:2 * NB]
        osems = rest[2 * NB:]
        wid = lax.axis_index("s") * info.num_cores + lax.axis_index("c")
        row0 = wid * nch
        pltpu.sync_copy(idx_hbm.at[pl.ds(row0, nch)], idx_v)
        gh = [None] * nch
        oh = [None] * nch
        for c in range(NB):
            gh[c] = pltpu.async_copy(table_hbm.at[idx_v.at[c]], bufs[c],
                                     gsems[c])
        for c in range(nch):
            b = c % NB
            gh[c].wait()
            oh[c] = pltpu.async_copy(
                bufs[b], out_hbm.at[pl.ds((row0 + c) * CH, CH)], osems[b])
            n = c + NB
            if n < nch:
                oh[c].wait()
                gh[n] = pltpu.async_copy(table_hbm.at[idx_v.at[n]], bufs[b],
                                         gsems[b])
        for c in range(max(0, nch - NB), nch):
            oh[c].wait()

    return k(table, idx2d)


def _sc_gather_nchw(table, idx2d, B, HW):
    K, C = table.shape
    R, CH = idx2d.shape
    info = plsc.get_sparse_core_info()
    NW = info.num_cores * info.num_subcores
    nch = R // NW
    tok_w = nch * CH
    L = info.num_lanes

    mesh = plsc.VectorSubcoreMesh(core_axis_name="c", subcore_axis_name="s")

    @functools.partial(
        pl.kernel,
        mesh=mesh,
        out_type=jax.ShapeDtypeStruct((B, C, HW), jnp.float32),
        compiler_params=pltpu.CompilerParams(use_tc_tiling_on_sc=False,
                                             needs_layout_passes=False),
        scratch_types=[
            pltpu.VMEM((nch, CH), jnp.int32),
            pltpu.VMEM((CH, C), jnp.float32),
            pltpu.VMEM((CH, C), jnp.float32),
            pltpu.VMEM((C, CH), jnp.float32),
            pltpu.SemaphoreType.DMA,
            pltpu.SemaphoreType.DMA,
            pltpu.SemaphoreType.DMA,
        ],
    )
    def k(table_hbm, idx_hbm, out_hbm, idx_v, buf0, buf1, buf_t,
          g0, g1, osem):
        bufs = (buf0, buf1)
        gsems = (g0, g1)
        wid = lax.axis_index("s") * info.num_cores + lax.axis_index("c")
        row0 = wid * nch
        tok0 = wid * tok_w
        pltpu.sync_copy(idx_hbm.at[pl.ds(row0, nch)], idx_v)
        lanes = lax.iota(jnp.int32, L)

        def transpose_chunk(src):
            def tbody(ch, _):
                ch_vec = jnp.full((L,), 0, jnp.int32) + ch
                for g in range(CH // L):
                    rows = lanes + (L * g)
                    v = plsc.load_gather(src, [rows, ch_vec])
                    plsc.store_scatter(buf_t, [ch_vec, rows], v)
                return 0
            lax.fori_loop(0, C, tbody, 0)

        gh = [None] * nch
        gh[0] = pltpu.async_copy(table_hbm.at[idx_v.at[0]], bufs[0],
                                 gsems[0])
        oh = None
        for c in range(nch):
            b = c % 2
            if c + 1 < nch:
                gh[c + 1] = pltpu.async_copy(
                    table_hbm.at[idx_v.at[c + 1]], bufs[(c + 1) % 2],
                    gsems[(c + 1) % 2])
            gh[c].wait()
            if oh is not None:
                oh.wait()
            transpose_chunk(bufs[b])
            tok = tok0 + c * CH
            bb = tok // HW
            hw0 = tok % HW
            oh = pltpu.async_copy(
                buf_t, out_hbm.at[bb, :, pl.ds(hw0, CH)], osem)
        oh.wait()

    return k(table, idx2d)


def kernel(x, e_i_ts):
    B, C, H, W = x.shape
    HW = H * W
    x_r = x.reshape(B, C, HW)
    table = e_i_ts.T
    total = B * C * HW
    idx3, loss_arr = _tc_stage(x_r, e_i_ts, total)
    q = _sc_gather_nchw(table, idx3.reshape(-1, 128), B, HW)
    q = q.reshape(B, C, H, W)
    loss = loss_arr[0, 0]
    return (q, loss, loss, idx3.reshape(B, HW))


# speedup vs baseline: 1.2334x; 1.2334x over previous
import functools

import jax
import jax.numpy as jnp
from jax import lax
from jax.experimental import pallas as pl
from jax.experimental.pallas import tpu as pltpu
from jax.experimental.pallas import tpu_sc as plsc


def _tc_stage(x_r, e, total_count):
    B, C, HW = x_r.shape
    K = e.shape[1]
    inv_count = 1.0 / total_count

    def body(x_ref, e_ref, idx_ref, loss_ref, acc_ref):
        i = pl.program_id(0)
        xb = x_ref[0]
        et = e_ref[...]
        x2 = jnp.sum(xb * xb, axis=0)
        e2 = jnp.sum(et * et, axis=0)
        xe = lax.dot_general(
            xb, et, (((0,), (0,)), ((), ())),
            preferred_element_type=jnp.float32)
        scores = (x2[:, None] - 2.0 * xe) + e2[None, :]
        mins = jnp.min(scores, axis=1)
        idx = jnp.argmin(scores, axis=1).astype(jnp.int32)
        idx_ref[0, 0, :] = idx

        @pl.when(i == 0)
        def _():
            acc_ref[...] = jnp.zeros_like(acc_ref)

        acc_ref[...] += mins.reshape(acc_ref.shape)

        @pl.when(i == pl.num_programs(0) - 1)
        def _():
            loss_ref[0, 0] = jnp.sum(acc_ref[...]) * inv_count

    return pl.pallas_call(
        body,
        grid=(B,),
        in_specs=[
            pl.BlockSpec((1, C, HW), lambda i: (i, 0, 0)),
            pl.BlockSpec((C, K), lambda i: (0, 0)),
        ],
        out_specs=[
            pl.BlockSpec((1, 1, HW), lambda i: (i, 0, 0)),
            pl.BlockSpec(block_shape=(1, 1), index_map=lambda i: (0, 0),
                         memory_space=pltpu.SMEM),
        ],
        out_shape=[
            jax.ShapeDtypeStruct((B, 1, HW), jnp.int32),
            jax.ShapeDtypeStruct((1, 1), jnp.float32),
        ],
        scratch_shapes=[pltpu.VMEM((8, HW // 8), jnp.float32)],
        compiler_params=pltpu.CompilerParams(
            dimension_semantics=("arbitrary",)),
    )(x_r, e)


def _sc_gather(table, idx2d):
    K, C = table.shape
    R, CH = idx2d.shape
    info = plsc.get_sparse_core_info()
    NW = info.num_cores * info.num_subcores
    nch = R // NW
    NB = min(3, nch)

    mesh = plsc.VectorSubcoreMesh(core_axis_name="c", subcore_axis_name="s")

    @functools.partial(
        pl.kernel,
        mesh=mesh,
        out_type=jax.ShapeDtypeStruct((R * CH, C), jnp.float32),
        scratch_types=[
            pltpu.VMEM((nch, CH), jnp.int32),
        ]
        + [pltpu.VMEM((CH, C), jnp.float32) for _ in range(NB)]
        + [pltpu.SemaphoreType.DMA for _ in range(2 * NB)],
    )
    def k(table_hbm, idx_hbm, out_hbm, idx_v, *rest):
        bufs = rest[:NB]
        gsems = rest[NB:2 * NB]
        osems = rest[2 * NB:]
        wid = lax.axis_index("s") * info.num_cores + lax.axis_index("c")
        row0 = wid * nch
        pltpu.sync_copy(idx_hbm.at[pl.ds(row0, nch)], idx_v)
        gh = [None] * nch
        oh = [None] * nch
        for c in range(NB):
            gh[c] = pltpu.async_copy(table_hbm.at[idx_v.at[c]], bufs[c],
                                     gsems[c])
        for c in range(nch):
            b = c % NB
            gh[c].wait()
            oh[c] = pltpu.async_copy(
                bufs[b], out_hbm.at[pl.ds((row0 + c) * CH, CH)], osems[b])
            n = c + NB
            if n < nch:
                oh[c].wait()
                gh[n] = pltpu.async_copy(table_hbm.at[idx_v.at[n]], bufs[b],
                                         gsems[b])
        for c in range(max(0, nch - NB), nch):
            oh[c].wait()

    return k(table, idx2d)


def _sc_gather_nchw(table, idx2d, B, HW):
    K, C = table.shape
    R, CH = idx2d.shape
    info = plsc.get_sparse_core_info()
    NW = info.num_cores * info.num_subcores
    nch = R // NW
    tok_w = nch * CH
    L = info.num_lanes

    mesh = plsc.VectorSubcoreMesh(core_axis_name="c", subcore_axis_name="s")

    @functools.partial(
        pl.kernel,
        mesh=mesh,
        out_type=jax.ShapeDtypeStruct((B, C, HW), jnp.float32),
        compiler_params=pltpu.CompilerParams(use_tc_tiling_on_sc=False,
                                             needs_layout_passes=False),
        scratch_types=[
            pltpu.VMEM((nch, CH), jnp.int32),
            pltpu.VMEM((CH, C), jnp.float32),
            pltpu.VMEM((CH, C), jnp.float32),
            pltpu.VMEM((C, CH), jnp.float32),
            pltpu.SemaphoreType.DMA,
            pltpu.SemaphoreType.DMA,
            pltpu.SemaphoreType.DMA,
        ],
    )
    def k(table_hbm, idx_hbm, out_hbm, idx_v, buf0, buf1, buf_t,
          g0, g1, osem):
        bufs = (buf0, buf1)
        gsems = (g0, g1)
        wid = lax.axis_index("s") * info.num_cores + lax.axis_index("c")
        row0 = wid * nch
        tok0 = wid * tok_w
        pltpu.sync_copy(idx_hbm.at[pl.ds(row0, nch)], idx_v)
        lanes = lax.iota(jnp.int32, L)

        def transpose_chunk(src):
            @plsc.parallel_loop(0, C, 1, unroll=4)
            def tbody(ch):
                ch_vec = jnp.full((L,), 0, jnp.int32) + ch
                for g in range(CH // L):
                    rows = lanes + (L * g)
                    v = plsc.load_gather(src, [rows, ch_vec])
                    plsc.store_scatter(buf_t, [ch_vec, rows], v)

        gh = [None] * nch
        gh[0] = pltpu.async_copy(table_hbm.at[idx_v.at[0]], bufs[0],
                                 gsems[0])
        oh = None
        for c in range(nch):
            b = c % 2
            if c + 1 < nch:
                gh[c + 1] = pltpu.async_copy(
                    table_hbm.at[idx_v.at[c + 1]], bufs[(c + 1) % 2],
                    gsems[(c + 1) % 2])
            gh[c].wait()
            if oh is not None:
                oh.wait()
            transpose_chunk(bufs[b])
            tok = tok0 + c * CH
            bb = tok // HW
            hw0 = tok % HW
            oh = pltpu.async_copy(
                buf_t, out_hbm.at[bb, :, pl.ds(hw0, CH)], osem)
        oh.wait()

    return k(table, idx2d)


def kernel(x, e_i_ts):
    B, C, H, W = x.shape
    HW = H * W
    x_r = x.reshape(B, C, HW)
    table = e_i_ts.T
    total = B * C * HW
    idx3, loss_arr = _tc_stage(x_r, e_i_ts, total)
    q = _sc_gather_nchw(table, idx3.reshape(-1, 128), B, HW)
    q = q.reshape(B, C, H, W)
    loss = loss_arr[0, 0]
    return (q, loss, loss, idx3.reshape(B, HW))


# speedup vs baseline: 2.4066x; 1.9512x over previous
import functools

import jax
import jax.numpy as jnp
from jax import lax
from jax.experimental import pallas as pl
from jax.experimental.pallas import tpu as pltpu
from jax.experimental.pallas import tpu_sc as plsc


def _tc_stage(x_r, e, total_count):
    B, C, HW = x_r.shape
    K = e.shape[1]
    inv_count = 1.0 / total_count

    def body(x_ref, e_ref, idx_ref, loss_ref, acc_ref):
        i = pl.program_id(0)
        xb = x_ref[0]
        et = e_ref[...]
        x2 = jnp.sum(xb * xb, axis=0)
        e2 = jnp.sum(et * et, axis=0)
        xe_t = lax.dot_general(
            et, xb, (((0,), (0,)), ((), ())),
            preferred_element_type=jnp.float32)
        scores_t = (x2[None, :] - 2.0 * xe_t) + e2[:, None]
        mins = jnp.min(scores_t, axis=0)
        idx = jnp.argmin(scores_t, axis=0).astype(jnp.int32)
        idx_ref[0, 0, :] = idx

        @pl.when(i == 0)
        def _():
            acc_ref[...] = jnp.zeros_like(acc_ref)

        acc_ref[...] += mins.reshape(acc_ref.shape)

        @pl.when(i == pl.num_programs(0) - 1)
        def _():
            loss_ref[0, 0] = jnp.sum(acc_ref[...]) * inv_count

    return pl.pallas_call(
        body,
        grid=(B,),
        in_specs=[
            pl.BlockSpec((1, C, HW), lambda i: (i, 0, 0)),
            pl.BlockSpec((C, K), lambda i: (0, 0)),
        ],
        out_specs=[
            pl.BlockSpec((1, 1, HW), lambda i: (i, 0, 0)),
            pl.BlockSpec(block_shape=(1, 1), index_map=lambda i: (0, 0),
                         memory_space=pltpu.SMEM),
        ],
        out_shape=[
            jax.ShapeDtypeStruct((B, 1, HW), jnp.int32),
            jax.ShapeDtypeStruct((1, 1), jnp.float32),
        ],
        scratch_shapes=[pltpu.VMEM((8, HW // 8), jnp.float32)],
        compiler_params=pltpu.CompilerParams(
            dimension_semantics=("arbitrary",)),
    )(x_r, e)


def _sc_gather(table, idx2d):
    K, C = table.shape
    R, CH = idx2d.shape
    info = plsc.get_sparse_core_info()
    NW = info.num_cores * info.num_subcores
    nch = R // NW
    NB = min(3, nch)

    mesh = plsc.VectorSubcoreMesh(core_axis_name="c", subcore_axis_name="s")

    @functools.partial(
        pl.kernel,
        mesh=mesh,
        out_type=jax.ShapeDtypeStruct((R * CH, C), jnp.float32),
        scratch_types=[
            pltpu.VMEM((nch, CH), jnp.int32),
        ]
        + [pltpu.VMEM((CH, C), jnp.float32) for _ in range(NB)]
        + [pltpu.SemaphoreType.DMA for _ in range(2 * NB)],
    )
    def k(table_hbm, idx_hbm, out_hbm, idx_v, *rest):
        bufs = rest[:NB]
        gsems = rest[NB:2 * NB]
        osems = rest[2 * NB:]
        wid = lax.axis_index("s") * info.num_cores + lax.axis_index("c")
        row0 = wid * nch
        pltpu.sync_copy(idx_hbm.at[pl.ds(row0, nch)], idx_v)
        gh = [None] * nch
        oh = [None] * nch
        for c in range(NB):
            gh[c] = pltpu.async_copy(table_hbm.at[idx_v.at[c]], bufs[c],
                                     gsems[c])
        for c in range(nch):
            b = c % NB
            gh[c].wait()
            oh[c] = pltpu.async_copy(
                bufs[b], out_hbm.at[pl.ds((row0 + c) * CH, CH)], osems[b])
            n = c + NB
            if n < nch:
                oh[c].wait()
                gh[n] = pltpu.async_copy(table_hbm.at[idx_v.at[n]], bufs[b],
                                         gsems[b])
        for c in range(max(0, nch - NB), nch):
            oh[c].wait()

    return k(table, idx2d)


def kernel(x, e_i_ts):
    B, C, H, W = x.shape
    HW = H * W
    x_r = x.reshape(B, C, HW)
    table = e_i_ts.T
    total = B * C * HW
    idx3, loss_arr = _tc_stage(x_r, e_i_ts, total)
    q_flat = _sc_gather(table, idx3.reshape(-1, 128))
    q = q_flat.reshape(B, H, W, C).transpose(0, 3, 1, 2)
    loss = loss_arr[0, 0]
    return (q, loss, loss, idx3.reshape(B, HW))
